# baseline (speedup 1.0000x reference)
import functools

import jax
import jax.numpy as jnp
from jax import lax
from jax.experimental import pallas as pl
from jax.experimental.pallas import tpu as pltpu
from jax.experimental.pallas import tpu_sc as plsc

VOCAB = 1000000
EMBED = 64
BATCH = 16384

_info = plsc.get_sparse_core_info()
_NC = _info.num_cores
_NS = _info.num_subcores
_NW = _NC * _NS
_BPW = BATCH // _NW

_mesh = plsc.VectorSubcoreMesh(core_axis_name="c", subcore_axis_name="s")


@functools.partial(
    pl.kernel,
    mesh=_mesh,
    out_type=(
        jax.ShapeDtypeStruct((BATCH, EMBED), jnp.float32),
        jax.ShapeDtypeStruct((BATCH, EMBED), jnp.float32),
    ),
    scratch_types=[
        pltpu.VMEM((_BPW,), jnp.int32),
        pltpu.VMEM((_BPW,), jnp.int32),
        pltpu.VMEM((_BPW, EMBED), jnp.float32),
        pltpu.VMEM((_BPW, EMBED), jnp.float32),
        pltpu.SemaphoreType.DMA,
        pltpu.SemaphoreType.DMA,
    ],
    compiler_params=pltpu.CompilerParams(use_tc_tiling_on_sc=False),
)
def _lookup(center_hbm, context_hbm, table_hbm, out_c_hbm, out_x_hbm,
            idx_c, idx_x, rows_c, rows_x, sem_c, sem_x):
    wid = lax.axis_index("s") * _NC + lax.axis_index("c")
    base = wid * _BPW
    pltpu.sync_copy(center_hbm.at[pl.ds(base, _BPW)], idx_c)
    pltpu.sync_copy(context_hbm.at[pl.ds(base, _BPW)], idx_x)
    cp_c = pltpu.async_copy(table_hbm.at[idx_c], rows_c, sem_c)
    cp_x = pltpu.async_copy(table_hbm.at[idx_x], rows_x, sem_x)
    cp_c.wait()
    pltpu.sync_copy(rows_c, out_c_hbm.at[pl.ds(base, _BPW)])
    cp_x.wait()
    pltpu.sync_copy(rows_x, out_x_hbm.at[pl.ds(base, _BPW)])


def kernel(center, context, word_embeds):
    return _lookup(center, context, word_embeds)
